# Initial kernel scaffold; baseline (speedup 1.0000x reference)
import functools

import jax
import jax.numpy as jnp
from jax import lax
from jax.experimental import pallas as pl
from jax.experimental.pallas import tpu as pltpu
from jax.experimental.pallas import tpu_sc as plsc

NUM_CORES = 2
NUM_SUBCORES = 16
NUM_WORKERS = NUM_CORES * NUM_SUBCORES
CHUNK = 128


def _emb_call(total, n_chunks, emb_dim):
    mesh = plsc.VectorSubcoreMesh(
        core_axis_name="c",
        subcore_axis_name="s",
        num_cores=NUM_CORES,
        num_subcores=NUM_SUBCORES,
    )
    per_w = n_chunks * CHUNK

    @functools.partial(
        pl.kernel,
        out_type=jax.ShapeDtypeStruct((total, emb_dim), jnp.float32),
        mesh=mesh,
        scratch_types=[
            pltpu.VMEM((n_chunks, CHUNK), jnp.int32),
            pltpu.VMEM((CHUNK, emb_dim), jnp.float32),
            pltpu.SemaphoreType.DMA,
        ],
    )
    def emb(x_hbm, tbl_hbm, out_hbm, idx_v, rows_v, gsem):
        wid = lax.axis_index("s") * NUM_CORES + lax.axis_index("c")
        base = wid * per_w
        pltpu.sync_copy(x_hbm.at[wid], idx_v)

        @pl.loop(0, n_chunks)
        def _chunk(j):
            pltpu.async_copy(tbl_hbm.at[idx_v.at[j]], rows_v, gsem).wait()
            pltpu.sync_copy(rows_v, out_hbm.at[pl.ds(base + j * CHUNK, CHUNK)])

    return emb


def kernel(x, table):
    b, s = x.shape
    v, d = table.shape
    total = b * s
    per_w = total // NUM_WORKERS
    n_chunks = per_w // CHUNK
    idx3 = x.reshape(NUM_WORKERS, n_chunks, CHUNK)
    out = _emb_call(total, n_chunks, d)(idx3, table)
    return out.reshape(b, s, d)


# speedup vs baseline: 4.0812x; 4.0812x over previous
import functools

import jax
import jax.numpy as jnp
from jax import lax
from jax.experimental import pallas as pl
from jax.experimental.pallas import tpu as pltpu
from jax.experimental.pallas import tpu_sc as plsc

NUM_CORES = 2
NUM_SUBCORES = 16
NUM_WORKERS = NUM_CORES * NUM_SUBCORES
CHUNK = 128


def _emb_call(total, n_chunks, emb_dim):
    mesh = plsc.VectorSubcoreMesh(
        core_axis_name="c",
        subcore_axis_name="s",
        num_cores=NUM_CORES,
        num_subcores=NUM_SUBCORES,
    )
    per_w = n_chunks * CHUNK

    @functools.partial(
        pl.kernel,
        out_type=jax.ShapeDtypeStruct((total, emb_dim), jnp.float32),
        mesh=mesh,
        compiler_params=pltpu.CompilerParams(use_tc_tiling_on_sc=False),
        scratch_types=[
            pltpu.VMEM((n_chunks, CHUNK), jnp.int32),
            pltpu.VMEM((CHUNK, emb_dim), jnp.float32),
            pltpu.SemaphoreType.DMA,
        ],
    )
    def emb(x_hbm, tbl_hbm, out_hbm, idx_v, rows_v, gsem):
        wid = lax.axis_index("s") * NUM_CORES + lax.axis_index("c")
        base = wid * per_w
        pltpu.sync_copy(x_hbm.at[wid], idx_v)

        @pl.loop(0, n_chunks)
        def _chunk(j):
            pltpu.async_copy(tbl_hbm.at[idx_v.at[j]], rows_v, gsem).wait()
            pltpu.sync_copy(rows_v, out_hbm.at[pl.ds(base + j * CHUNK, CHUNK)])

    return emb


def kernel(x, table):
    b, s = x.shape
    v, d = table.shape
    total = b * s
    per_w = total // NUM_WORKERS
    n_chunks = per_w // CHUNK
    idx3 = x.reshape(NUM_WORKERS, n_chunks, CHUNK)
    out = _emb_call(total, n_chunks, d)(idx3, table)
    return out.reshape(b, s, d)


# speedup vs baseline: 4.6827x; 1.1474x over previous
import functools

import jax
import jax.numpy as jnp
from jax import lax
from jax.experimental import pallas as pl
from jax.experimental.pallas import tpu as pltpu
from jax.experimental.pallas import tpu_sc as plsc

NUM_CORES = 2
NUM_SUBCORES = 16
NUM_WORKERS = NUM_CORES * NUM_SUBCORES
CHUNK = 128
NBUF = 5


def _emb_call(total, n_chunks, emb_dim):
    mesh = plsc.VectorSubcoreMesh(
        core_axis_name="c",
        subcore_axis_name="s",
        num_cores=NUM_CORES,
        num_subcores=NUM_SUBCORES,
    )
    per_w = n_chunks * CHUNK

    @functools.partial(
        pl.kernel,
        out_type=jax.ShapeDtypeStruct((total, emb_dim), jnp.float32),
        mesh=mesh,
        compiler_params=pltpu.CompilerParams(use_tc_tiling_on_sc=False),
        scratch_types=[
            pltpu.VMEM((n_chunks, CHUNK), jnp.int32),
            pltpu.VMEM((NBUF, CHUNK, emb_dim), jnp.float32),
            [pltpu.SemaphoreType.DMA] * NBUF,
        ],
    )
    def emb(x_hbm, tbl_hbm, out_hbm, idx_v, rows_v, gsems):
        wid = lax.axis_index("s") * NUM_CORES + lax.axis_index("c")
        base = wid * per_w
        pltpu.sync_copy(x_hbm.at[wid], idx_v)

        for b in range(NBUF):
            pltpu.async_copy(tbl_hbm.at[idx_v.at[b]], rows_v.at[b], gsems[b])

        @pl.loop(0, n_chunks, step=NBUF)
        def _round(j):
            for b in range(NBUF):
                k = j + b
                pltpu.make_async_copy(
                    tbl_hbm.at[idx_v.at[k]], rows_v.at[b], gsems[b]
                ).wait()
                pltpu.sync_copy(
                    rows_v.at[b], out_hbm.at[pl.ds(base + k * CHUNK, CHUNK)]
                )

                @pl.when(k + NBUF < n_chunks)
                def _():
                    pltpu.async_copy(
                        tbl_hbm.at[idx_v.at[k + NBUF]], rows_v.at[b], gsems[b]
                    )

    return emb


def kernel(x, table):
    b, s = x.shape
    v, d = table.shape
    total = b * s
    per_w = total // NUM_WORKERS
    n_chunks = per_w // CHUNK
    idx3 = x.reshape(NUM_WORKERS, n_chunks, CHUNK)
    out = _emb_call(total, n_chunks, d)(idx3, table)
    return out.reshape(b, s, d)
